# baseline (speedup 1.0000x reference)
import jax
import jax.numpy as jnp
from jax.experimental import pallas as pl

N_NODES = 10000


def _fused_mlp_kernel(x_ref, a_ref, b_ref, bo2_ref, out_ref):
    h = jnp.dot(x_ref[...], a_ref[0:128, :], preferred_element_type=jnp.float32)
    h = jnp.maximum(h + a_ref[384:385, :], 0.0)
    h = jnp.dot(h, a_ref[128:256, :], preferred_element_type=jnp.float32)
    h = jnp.maximum(h + a_ref[392:393, :], 0.0)

    g = jnp.sum(h, axis=0, keepdims=True) * (1.0 / N_NODES)
    g = (jnp.dot(g, a_ref[256:384, :], preferred_element_type=jnp.float32)
         + a_ref[400:401, :])
    p = (jnp.dot(g, b_ref[0:128, :], preferred_element_type=jnp.float32)
         + b_ref[128:129, :])
    p = jnp.maximum(p, 0.0)
    out = jnp.sum(p * b_ref[136:137, :]) + bo2_ref[0, 0]
    out_ref[...] = out.reshape(1, 1)


def kernel(x, edge_index, W1, b1, W2, b2, W3, b3, Wo1, bo1, Wo2, bo2):
    del edge_index
    tile8 = lambda v: jnp.tile(v.reshape(1, -1), (8, 1))
    packA = jnp.concatenate(
        [W1, W2, W3, tile8(b1), tile8(b2), tile8(b3)], axis=0)
    packB = jnp.concatenate(
        [Wo1, tile8(bo1), tile8(Wo2.reshape(1, 64))], axis=0)

    out = pl.pallas_call(
        _fused_mlp_kernel,
        out_shape=jax.ShapeDtypeStruct((1, 1), jnp.float32),
    )(x, packA, packB, bo2.reshape(1, 1))
    return out


# speedup vs baseline: 1.5353x; 1.5353x over previous
import jax
import jax.numpy as jnp
from jax.experimental import pallas as pl

N_NODES = 10000


def _fused_mlp_kernel(x_ref, W1_ref, b1_ref, W2_ref, b2_ref, W3_ref, b3_ref,
                      Wo1_ref, bo1_ref, Wo2_ref, bo2_ref, out_ref):
    W3o1 = jnp.dot(W3_ref[...], Wo1_ref[...], preferred_element_type=jnp.float32)
    b3o1 = (jnp.dot(b3_ref[...], Wo1_ref[...], preferred_element_type=jnp.float32)
            + bo1_ref[...])

    h = jnp.dot(x_ref[...], W1_ref[...], preferred_element_type=jnp.float32)
    h = jnp.maximum(h + b1_ref[...], 0.0)
    h = jnp.dot(h, W2_ref[...], preferred_element_type=jnp.float32)
    h = jnp.maximum(h + b2_ref[...], 0.0)

    g = jnp.sum(h, axis=0, keepdims=True) * (1.0 / N_NODES)
    p = jnp.dot(g, W3o1, preferred_element_type=jnp.float32) + b3o1
    p = jnp.maximum(p, 0.0)
    out = jnp.sum(p * Wo2_ref[...].reshape(1, -1)) + bo2_ref[0, 0]
    out_ref[...] = out.reshape(1, 1)


def kernel(x, edge_index, W1, b1, W2, b2, W3, b3, Wo1, bo1, Wo2, bo2):
    del edge_index
    b1 = b1.reshape(1, -1)
    b2 = b2.reshape(1, -1)
    b3 = b3.reshape(1, -1)
    bo1 = bo1.reshape(1, -1)
    bo2 = bo2.reshape(1, -1)

    out = pl.pallas_call(
        _fused_mlp_kernel,
        out_shape=jax.ShapeDtypeStruct((1, 1), jnp.float32),
    )(x, W1, b1, W2, b2, W3, b3, Wo1, bo1, Wo2, bo2)
    return out
